# Initial kernel scaffold; baseline (speedup 1.0000x reference)
import jax
import jax.numpy as jnp
from jax.experimental import pallas as pl


def kernel(x, edge_index, W1, b1, W2, b2, Wa, ba, Wb, bb):
    raise NotImplementedError("write your pallas kernel here")


# speedup vs baseline: 3.7089x; 3.7089x over previous
import functools

import jax
import jax.numpy as jnp
from jax import lax
from jax.experimental import pallas as pl
from jax.experimental.pallas import tpu as pltpu
from jax.experimental.pallas import tpu_sc as plsc

_N = 10000
_E = 320000
_D = 128
_H = 256
_C = 40

_NC = 2
_NS = 16
_NW = _NC * _NS
_L = 16

_CHUNK = 80
_STAGE = 2000
_SCH = _STAGE // _CHUNK
_EPT1 = _E // _NS
_NST1 = _EPT1 // _STAGE
_EPT2 = _E // _NW
_NCH2 = _EPT2 // _CHUNK
_NP = 10240
_RPT = _NP // _NS
_ZROWS = 64

_BLK = 2000
_NBLK = 2048
_EBLK = 2000


def _n1_body(x_ref, w1_ref, b1_ref, t_ref):
    xb = x_ref[...]
    w1 = w1_ref[...]
    a = jnp.dot(xb, w1[:_D, :], preferred_element_type=jnp.float32)
    b = jnp.dot(xb, w1[_D:, :], preferred_element_type=jnp.float32) + b1_ref[...]
    t_ref[0] = a[:, :_D]
    t_ref[1] = a[:, _D:]
    t_ref[2] = b[:, :_D]
    t_ref[3] = b[:, _D:]


def _node1(x, W1, b1):
    return pl.pallas_call(
        _n1_body,
        grid=(_N // _BLK,),
        in_specs=[
            pl.BlockSpec((_BLK, _D), lambda i: (i, 0)),
            pl.BlockSpec((2 * _D, _H), lambda i: (0, 0)),
            pl.BlockSpec((1, _H), lambda i: (0, 0)),
        ],
        out_specs=pl.BlockSpec((4, _BLK, _D), lambda i: (0, i, 0)),
        out_shape=jax.ShapeDtypeStruct((4, _N, _D), jnp.float32),
    )(x, W1, b1)


def _sc1_body(t_hbm, src_hbm, dst_hbm, agg_hbm,
              sbuf, dbuf, bbuf, didxc,
              bufa, bufb, zrow, agg_sh, sem_a, sem_b):
    c = lax.axis_index("c")
    s = lax.axis_index("s")
    ebase = s * _EPT1

    def _zfill(i, _):
        for f in range(_D // _L):
            zrow[i, pl.ds(f * _L, _L)] = jnp.zeros((_L,), jnp.float32)
        return 0
    lax.fori_loop(0, _ZROWS, _zfill, 0)

    def _zcopy(k, _):
        pltpu.sync_copy(zrow, agg_sh.at[pl.ds(s * _RPT + k * _ZROWS, _ZROWS)])
        return 0
    lax.fori_loop(0, _RPT // _ZROWS, _zcopy, 0)

    plsc.subcore_barrier()

    aoff = c * _N
    boff = (2 + c) * _N

    def _stage(k, _):
        sbase = ebase + k * _STAGE
        pltpu.sync_copy(src_hbm.at[pl.ds(sbase, _STAGE)], sbuf)
        pltpu.sync_copy(dst_hbm.at[pl.ds(sbase, _STAGE)], dbuf)

        def _adj(i, _):
            sl = pl.ds(i * _L, _L)
            sbuf[sl] = sbuf[sl] + aoff
            bbuf[sl] = dbuf[sl] + boff
            return 0
        lax.fori_loop(0, _STAGE // _L, _adj, 0)

        def _chunk(i, _):
            base = i * _CHUNK
            sl = pl.ds(base, _CHUNK)
            for f in range(_CHUNK // _L):
                didxc[pl.ds(f * _L, _L)] = dbuf[pl.ds(base + f * _L, _L)]
            cp_a = pltpu.async_copy(t_hbm.at[sbuf.at[sl]], bufa, sem_a)
            cp_b = pltpu.async_copy(t_hbm.at[bbuf.at[sl]], bufb, sem_b)
            cp_a.wait()
            cp_b.wait()

            def _edge(j, _):
                for f in range(_D // _L):
                    fsl = pl.ds(f * _L, _L)
                    bufb[j, fsl] = jnp.maximum(bufa[j, fsl] + bufb[j, fsl], 0.0)
                return 0
            lax.fori_loop(0, _CHUNK, _edge, 0)
            pltpu.sync_copy(bufb, agg_sh.at[didxc], add=True)
            return 0
        lax.fori_loop(0, _SCH, _chunk, 0)
        return 0
    lax.fori_loop(0, _NST1, _stage, 0)

    plsc.subcore_barrier()

    def _cout(k, _):
        rows = pl.ds(s * _RPT + k * _ZROWS, _ZROWS)
        pltpu.sync_copy(agg_sh.at[rows], zrow)
        pltpu.sync_copy(zrow, agg_hbm.at[c].at[rows])
        return 0
    lax.fori_loop(0, _RPT // _ZROWS, _cout, 0)


def _sc_agg(t, src, dst):
    mesh = plsc.VectorSubcoreMesh(core_axis_name="c", subcore_axis_name="s",
                                  num_cores=_NC, num_subcores=_NS)
    f = pl.kernel(
        _sc1_body,
        out_type=jax.ShapeDtypeStruct((_NC, _NP, _D), jnp.float32),
        mesh=mesh,
        scratch_types=[
            pltpu.VMEM((_STAGE,), jnp.int32),
            pltpu.VMEM((_STAGE,), jnp.int32),
            pltpu.VMEM((_STAGE,), jnp.int32),
            pltpu.VMEM((_CHUNK,), jnp.int32),
            pltpu.VMEM((_CHUNK, _D), jnp.float32),
            pltpu.VMEM((_CHUNK, _D), jnp.float32),
            pltpu.VMEM((_ZROWS, _D), jnp.float32),
            pltpu.VMEM_SHARED((_NP, _D), jnp.float32),
            pltpu.SemaphoreType.DMA,
            pltpu.SemaphoreType.DMA,
        ],
    )
    return f(t, src, dst)


def _n2_body(agg_ref, w2_ref, b2_ref, wa_ref, ba_ref, pq_ref):
    w2 = w2_ref[...]
    wa = wa_ref[...]
    h = jnp.dot(agg_ref[0], w2[:_D, :], preferred_element_type=jnp.float32)
    h = h + jnp.dot(agg_ref[1], w2[_D:, :], preferred_element_type=jnp.float32)
    h = jnp.maximum(h + b2_ref[...], 0.0)
    pq_ref[0] = jnp.dot(h, wa[:_D, :], preferred_element_type=jnp.float32)
    pq_ref[1] = jnp.dot(h, wa[_D:, :], preferred_element_type=jnp.float32) + ba_ref[...]


def _node2(agg, W2, b2, Wa, ba):
    return pl.pallas_call(
        _n2_body,
        grid=(_NP // _NBLK,),
        in_specs=[
            pl.BlockSpec((2, _NBLK, _D), lambda i: (0, i, 0)),
            pl.BlockSpec((_H, _D), lambda i: (0, 0)),
            pl.BlockSpec((1, _D), lambda i: (0, 0)),
            pl.BlockSpec((_H, _D), lambda i: (0, 0)),
            pl.BlockSpec((1, _D), lambda i: (0, 0)),
        ],
        out_specs=pl.BlockSpec((2, _NBLK, _D), lambda i: (0, i, 0)),
        out_shape=jax.ShapeDtypeStruct((2, _NP, _D), jnp.float32),
    )(agg, W2, b2, Wa, ba)


def _sc2_body(pq_hbm, src_hbm, dst_hbm, he_hbm,
              aidx, bidx, bufa, bufb, sem_a, sem_b):
    c = lax.axis_index("c")
    s = lax.axis_index("s")
    wid = s * _NC + c
    ebase = wid * _EPT2

    pltpu.sync_copy(src_hbm.at[pl.ds(ebase, _EPT2)], aidx)
    pltpu.sync_copy(dst_hbm.at[pl.ds(ebase, _EPT2)], bidx)

    def _adj(i, _):
        sl = pl.ds(i * _L, _L)
        bidx[sl] = bidx[sl] + _NP
        return 0
    lax.fori_loop(0, _EPT2 // _L, _adj, 0)

    def _chunk(i, _):
        base = i * _CHUNK
        sl = pl.ds(base, _CHUNK)
        cp_a = pltpu.async_copy(pq_hbm.at[aidx.at[sl]], bufa, sem_a)
        cp_b = pltpu.async_copy(pq_hbm.at[bidx.at[sl]], bufb, sem_b)
        cp_a.wait()
        cp_b.wait()

        def _edge(j, _):
            for f in range(_D // _L):
                fsl = pl.ds(f * _L, _L)
                bufb[j, fsl] = jnp.maximum(bufa[j, fsl] + bufb[j, fsl], 0.0)
            return 0
        lax.fori_loop(0, _CHUNK, _edge, 0)
        pltpu.sync_copy(bufb, he_hbm.at[pl.ds(ebase + base, _CHUNK)])
        return 0
    lax.fori_loop(0, _NCH2, _chunk, 0)


def _sc_head_edges(pq, src, dst):
    mesh = plsc.VectorSubcoreMesh(core_axis_name="c", subcore_axis_name="s",
                                  num_cores=_NC, num_subcores=_NS)
    f = pl.kernel(
        _sc2_body,
        out_type=jax.ShapeDtypeStruct((_E, _D), jnp.float32),
        mesh=mesh,
        scratch_types=[
            pltpu.VMEM((_EPT2,), jnp.int32),
            pltpu.VMEM((_EPT2,), jnp.int32),
            pltpu.VMEM((_CHUNK, _D), jnp.float32),
            pltpu.VMEM((_CHUNK, _D), jnp.float32),
            pltpu.SemaphoreType.DMA,
            pltpu.SemaphoreType.DMA,
        ],
    )
    return f(pq, src, dst)


def _head_body(he_ref, wb_ref, bb_ref, out_ref):
    logits = jnp.dot(he_ref[...], wb_ref[...], preferred_element_type=jnp.float32)
    logits = logits + bb_ref[...]
    col = lax.broadcasted_iota(jnp.int32, logits.shape, 1)
    logits = jnp.where(col < _C, logits, -1e30)
    m = jnp.max(logits, axis=1, keepdims=True)
    z = jnp.exp(logits - m)
    lse = m + jnp.log(jnp.sum(z, axis=1, keepdims=True))
    out_ref[...] = (logits - lse)[:, :_C]


def _head(he, Wbp, bbp):
    return pl.pallas_call(
        _head_body,
        grid=(_E // _EBLK,),
        in_specs=[
            pl.BlockSpec((_EBLK, _D), lambda i: (i, 0)),
            pl.BlockSpec((_D, _D), lambda i: (0, 0)),
            pl.BlockSpec((1, _D), lambda i: (0, 0)),
        ],
        out_specs=pl.BlockSpec((_EBLK, _C), lambda i: (i, 0)),
        out_shape=jax.ShapeDtypeStruct((_E, _C), jnp.float32),
    )(he, Wbp, bbp)


def kernel(x, edge_index, W1, b1, W2, b2, Wa, ba, Wb, bb):
    src = edge_index[0]
    dst = edge_index[1]
    t = _node1(x, W1, b1.reshape(1, _H)).reshape(4 * _N, _D)
    agg = _sc_agg(t, src, dst)
    pq = _node2(agg, W2, b2.reshape(1, _D), Wa, ba.reshape(1, _D))
    he = _sc_head_edges(pq.reshape(2 * _NP, _D), src, dst)
    wbp = jnp.zeros((_D, _D), jnp.float32).at[:, :_C].set(Wb)
    bbp = jnp.zeros((1, _D), jnp.float32).at[:, :_C].set(bb)
    return _head(he, wbp, bbp)
